# baseline (speedup 1.0000x reference)
import functools

import jax
import jax.numpy as jnp
from jax import lax
from jax.experimental import pallas as pl
from jax.experimental.pallas import tpu as pltpu
from jax.experimental.pallas import tpu_sc as plsc

N_TOK = 4096
D_MODEL = 1024
D_FF = 4096
N_EXPERTS = 8
TOP_K = 2
TILE = 128
T_MAX = (N_TOK * TOP_K) // TILE + N_EXPERTS
PAD_ROWS = T_MAX * TILE
FSPLIT = 2
FB = D_FF // FSPLIT

NW = 32
SLOTS_W = (N_TOK * TOP_K) // NW
DSUB = 64
NDSUB = SLOTS_W // DSUB
TOKS_W = N_TOK // NW
CSUB = 32
NCSUB = TOKS_W // CSUB


def _route_body(x_ref, wg_ref, pos_ref, gates_ref, te_ref, nv_ref,
                ohs_ref, cum_ref):
    E = N_EXPERTS
    f32 = jnp.float32
    logits = jnp.dot(x_ref[...], wg_ref[...], preferred_element_type=f32)
    m = jnp.max(logits, axis=1, keepdims=True)
    ex = jnp.exp(logits - m)
    p = ex / jnp.sum(ex, axis=1, keepdims=True)
    idx8 = lax.broadcasted_iota(jnp.int32, (N_TOK, E), 1)
    g1 = jnp.max(p, axis=1, keepdims=True)
    e1 = jnp.min(jnp.where(p >= g1, idx8, E), axis=1, keepdims=True)
    pm = jnp.where(idx8 == e1, -1.0, p)
    g2 = jnp.max(pm, axis=1, keepdims=True)
    e2 = jnp.min(jnp.where(pm >= g2, idx8, E), axis=1, keepdims=True)
    oh1 = (idx8 == e1).astype(f32)
    oh2 = (idx8 == e2).astype(f32)
    ohs_ref[...] = oh1 + oh2
    r_i = lax.broadcasted_iota(jnp.int32, (128, 128), 0)
    c_i = lax.broadcasted_iota(jnp.int32, (128, 128), 1)
    ltri = (r_i >= c_i).astype(f32)

    def body(b, carry):
        off = pl.multiple_of(b * 128, 128)
        blk = ohs_ref[pl.ds(off, 128), :]
        incl = jnp.dot(ltri, blk, preferred_element_type=f32)
        cum_ref[pl.ds(off, 128), :] = incl + carry
        return carry + jnp.sum(blk, axis=0, keepdims=True)

    lax.fori_loop(0, N_TOK // 128, body, jnp.zeros((1, E), f32))
    cum = cum_ref[...]
    ohs = ohs_ref[...]
    excl = cum - ohs
    counts = cum[N_TOK - 1:, :]
    segpad = jnp.floor((counts + (TILE - 1)) / TILE) * TILE
    tri8 = (lax.broadcasted_iota(jnp.int32, (E, E), 0)
            <= lax.broadcasted_iota(jnp.int32, (E, E), 1)).astype(f32)
    cum8 = jnp.dot(segpad, tri8, preferred_element_type=f32)
    padoff = cum8 - segpad
    pos0 = jnp.sum(oh1 * (excl + padoff), axis=1, keepdims=True)
    pos1 = jnp.sum(oh2 * (excl + padoff), axis=1, keepdims=True)
    pos_ref[...] = jnp.concatenate([pos0, pos1], axis=1).astype(jnp.int32)
    gates_ref[...] = jnp.concatenate([g1, g2], axis=1)
    tend = cum8 / TILE
    tv = lax.broadcasted_iota(jnp.int32, (T_MAX, E), 0).astype(f32)
    te = jnp.sum((tv >= tend).astype(f32), axis=1, keepdims=True)
    te_ref[...] = jnp.minimum(te, E - 1).astype(jnp.int32)
    nv_ref[...] = (jnp.sum(segpad, axis=1, keepdims=True) / TILE).astype(jnp.int32)


def _routing(x, Wg):
    return pl.pallas_call(
        _route_body,
        out_shape=(
            jax.ShapeDtypeStruct((N_TOK, TOP_K), jnp.int32),
            jax.ShapeDtypeStruct((N_TOK, TOP_K), jnp.float32),
            jax.ShapeDtypeStruct((T_MAX, 1), jnp.int32),
            jax.ShapeDtypeStruct((1, 1), jnp.int32),
        ),
        scratch_shapes=[
            pltpu.VMEM((N_TOK, N_EXPERTS), jnp.float32),
            pltpu.VMEM((N_TOK, N_EXPERTS), jnp.float32),
        ],
    )(x, Wg)


_SC_MESH = plsc.VectorSubcoreMesh(core_axis_name="c", subcore_axis_name="s")


@functools.partial(
    pl.kernel,
    mesh=_SC_MESH,
    out_type=[
        jax.ShapeDtypeStruct((PAD_ROWS, D_MODEL), jnp.float32),
        jax.ShapeDtypeStruct((PAD_ROWS,), jnp.float32),
    ],
    scratch_types=[
        pltpu.VMEM((NDSUB, DSUB), jnp.int32),
        pltpu.VMEM((NDSUB, DSUB), jnp.int32),
        pltpu.VMEM((NDSUB, DSUB), jnp.float32),
        pltpu.VMEM((DSUB, D_MODEL), jnp.float32),
        pltpu.SemaphoreType.DMA,
        pltpu.SemaphoreType.DMA,
    ],
)
def _dispatch(x_hbm, pos_hbm, tok_hbm, g_hbm, xs_hbm, gs_hbm,
              pos_v, tok_v, g_v, rows_v, sem1, sem2):
    wid = lax.axis_index("s") * 2 + lax.axis_index("c")
    pltpu.sync_copy(pos_hbm.at[wid], pos_v)
    pltpu.sync_copy(tok_hbm.at[wid], tok_v)
    pltpu.sync_copy(g_hbm.at[wid], g_v)
    for j in range(NDSUB):
        pltpu.async_copy(x_hbm.at[tok_v.at[j]], rows_v, sem1).wait()
        pltpu.async_copy(rows_v, xs_hbm.at[pos_v.at[j]], sem2).wait()
        pltpu.async_copy(g_v.at[j], gs_hbm.at[pos_v.at[j]], sem2).wait()


def _ffn_pass1_body(te_ref, nv_ref, xs_ref, w1_ref, b1_ref, w2_ref, out_ref):
    t = pl.program_id(0)

    @pl.when(t < nv_ref[0])
    def _():
        h = jnp.dot(xs_ref[...], w1_ref[0], preferred_element_type=jnp.float32)
        h = jax.nn.gelu(h + b1_ref[0])
        out_ref[...] = jnp.dot(h, w2_ref[0], preferred_element_type=jnp.float32)


def _ffn_pass2_body(te_ref, nv_ref, xs_ref, w1_ref, b1_ref, w2_ref, b2_ref,
                    gs_ref, pin_ref, out_ref):
    t = pl.program_id(0)

    @pl.when(t < nv_ref[0])
    def _():
        h = jnp.dot(xs_ref[...], w1_ref[0], preferred_element_type=jnp.float32)
        h = jax.nn.gelu(h + b1_ref[0])
        o = jnp.dot(h, w2_ref[0], preferred_element_type=jnp.float32)
        o = pin_ref[...] + o + b2_ref[0]
        out_ref[...] = o * gs_ref[0, 0][:, None]


def _grouped_ffn(xs, W1, b1, W2, b2, gs, tile_expert, nvalid):
    gs3 = gs.reshape(T_MAX, 1, TILE)
    b1r = b1.reshape(N_EXPERTS, 1, D_FF)
    b2r = b2.reshape(N_EXPERTS, 1, D_MODEL)
    xs_spec = pl.BlockSpec((TILE, D_MODEL), lambda t, te, nv: (t, 0))
    out_spec = pl.BlockSpec((TILE, D_MODEL), lambda t, te, nv: (t, 0))

    def wspecs(fh):
        return [
            pl.BlockSpec((1, D_MODEL, FB), lambda t, te, nv: (te[t], 0, fh)),
            pl.BlockSpec((1, 1, FB), lambda t, te, nv: (te[t], 0, fh)),
            pl.BlockSpec((1, FB, D_MODEL), lambda t, te, nv: (te[t], fh, 0)),
        ]

    out_sds = jax.ShapeDtypeStruct((PAD_ROWS, D_MODEL), jnp.float32)
    partial1 = pl.pallas_call(
        _ffn_pass1_body,
        grid_spec=pltpu.PrefetchScalarGridSpec(
            num_scalar_prefetch=2,
            grid=(T_MAX,),
            in_specs=[xs_spec] + wspecs(0),
            out_specs=out_spec,
        ),
        out_shape=out_sds,
    )(tile_expert, nvalid, xs, W1, b1r, W2)
    return pl.pallas_call(
        _ffn_pass2_body,
        grid_spec=pltpu.PrefetchScalarGridSpec(
            num_scalar_prefetch=2,
            grid=(T_MAX,),
            in_specs=[xs_spec] + wspecs(1) + [
                pl.BlockSpec((1, 1, D_MODEL), lambda t, te, nv: (te[t], 0, 0)),
                pl.BlockSpec((1, 1, TILE), lambda t, te, nv: (t, 0, 0)),
                pl.BlockSpec((TILE, D_MODEL), lambda t, te, nv: (t, 0)),
            ],
            out_specs=out_spec,
        ),
        out_shape=out_sds,
    )(tile_expert, nvalid, xs, W1, b1r, W2, b2r, gs3, partial1)


@functools.partial(
    pl.kernel,
    mesh=_SC_MESH,
    out_type=jax.ShapeDtypeStruct((N_TOK, D_MODEL), jnp.float32),
    scratch_types=[
        pltpu.VMEM((NCSUB, CSUB), jnp.int32),
        pltpu.VMEM((NCSUB, CSUB), jnp.int32),
        pltpu.VMEM((CSUB, D_MODEL), jnp.float32),
        pltpu.VMEM((CSUB, D_MODEL), jnp.float32),
        pltpu.SemaphoreType.DMA,
        pltpu.SemaphoreType.DMA,
    ],
)
def _combine(outg_hbm, pe_hbm, po_hbm, y_hbm,
             pe_v, po_v, a_v, b_v, sem1, sem2):
    wid = lax.axis_index("s") * 2 + lax.axis_index("c")
    pltpu.sync_copy(pe_hbm.at[wid], pe_v)
    pltpu.sync_copy(po_hbm.at[wid], po_v)
    for j in range(NCSUB):
        cp1 = pltpu.async_copy(outg_hbm.at[pe_v.at[j]], a_v, sem1)
        cp2 = pltpu.async_copy(outg_hbm.at[po_v.at[j]], b_v, sem2)
        cp1.wait()
        cp2.wait()

        def col_body(c, _, r):
            base = c * 64
            for u in range(4):
                sl = pl.ds(base + u * 16, 16)
                a_v[r, sl] = a_v[r, sl] + b_v[r, sl]
            return 0

        def row_body(r, _):
            lax.fori_loop(0, D_MODEL // 64,
                          functools.partial(col_body, r=r), 0)
            return 0

        lax.fori_loop(0, CSUB, row_body, 0)
        pltpu.sync_copy(a_v, y_hbm.at[pl.ds(wid * TOKS_W + j * CSUB, CSUB)])


def kernel(x, Wg, W1, b1, W2, b2):
    pos2, gates2, te, nv = _routing(x, Wg)
    te_flat = te.reshape(T_MAX)
    nv_flat = nv.reshape(1)
    pos3 = pos2.reshape(NW, NDSUB, DSUB)
    tok3 = (jnp.arange(N_TOK * TOP_K, dtype=jnp.int32) // TOP_K).reshape(
        NW, NDSUB, DSUB)
    g3 = gates2.reshape(NW, NDSUB, DSUB)
    xs, gsort = _dispatch(x, pos3, tok3, g3)
    outg = _grouped_ffn(xs, W1, b1, W2, b2, gsort, te_flat, nv_flat)
    pe = pos2[:, 0].reshape(NW, NCSUB, CSUB)
    po = pos2[:, 1].reshape(NW, NCSUB, CSUB)
    y = _combine(outg, pe, po)
    return y


# speedup vs baseline: 1.0702x; 1.0702x over previous
import functools

import jax
import jax.numpy as jnp
from jax import lax
from jax.experimental import pallas as pl
from jax.experimental.pallas import tpu as pltpu
from jax.experimental.pallas import tpu_sc as plsc

N_TOK = 4096
D_MODEL = 1024
D_FF = 4096
N_EXPERTS = 8
TOP_K = 2
TILE = 256
T_MAX = (N_TOK * TOP_K) // TILE + N_EXPERTS
PAD_ROWS = T_MAX * TILE
FSPLIT = 2
FB = D_FF // FSPLIT

NW = 32
SLOTS_W = (N_TOK * TOP_K) // NW
DSUB = 64
NDSUB = SLOTS_W // DSUB
TOKS_W = N_TOK // NW
CSUB = 32
NCSUB = TOKS_W // CSUB


def _route_body(x_ref, wg_ref, pos_ref, gates_ref, te_ref, nv_ref,
                ohs_ref, cum_ref):
    E = N_EXPERTS
    f32 = jnp.float32
    logits = jnp.dot(x_ref[...], wg_ref[...], preferred_element_type=f32)
    m = jnp.max(logits, axis=1, keepdims=True)
    ex = jnp.exp(logits - m)
    p = ex / jnp.sum(ex, axis=1, keepdims=True)
    idx8 = lax.broadcasted_iota(jnp.int32, (N_TOK, E), 1)
    g1 = jnp.max(p, axis=1, keepdims=True)
    e1 = jnp.min(jnp.where(p >= g1, idx8, E), axis=1, keepdims=True)
    pm = jnp.where(idx8 == e1, -1.0, p)
    g2 = jnp.max(pm, axis=1, keepdims=True)
    e2 = jnp.min(jnp.where(pm >= g2, idx8, E), axis=1, keepdims=True)
    oh1 = (idx8 == e1).astype(f32)
    oh2 = (idx8 == e2).astype(f32)
    ohs_ref[...] = oh1 + oh2
    r_i = lax.broadcasted_iota(jnp.int32, (128, 128), 0)
    c_i = lax.broadcasted_iota(jnp.int32, (128, 128), 1)
    ltri = (r_i >= c_i).astype(f32)

    def body(b, carry):
        off = pl.multiple_of(b * 128, 128)
        blk = ohs_ref[pl.ds(off, 128), :]
        incl = jnp.dot(ltri, blk, preferred_element_type=f32)
        cum_ref[pl.ds(off, 128), :] = incl + carry
        return carry + jnp.sum(blk, axis=0, keepdims=True)

    lax.fori_loop(0, N_TOK // 128, body, jnp.zeros((1, E), f32))
    cum = cum_ref[...]
    ohs = ohs_ref[...]
    excl = cum - ohs
    counts = cum[N_TOK - 1:, :]
    segpad = jnp.floor((counts + (TILE - 1)) / TILE) * TILE
    tri8 = (lax.broadcasted_iota(jnp.int32, (E, E), 0)
            <= lax.broadcasted_iota(jnp.int32, (E, E), 1)).astype(f32)
    cum8 = jnp.dot(segpad, tri8, preferred_element_type=f32)
    padoff = cum8 - segpad
    pos0 = jnp.sum(oh1 * (excl + padoff), axis=1, keepdims=True)
    pos1 = jnp.sum(oh2 * (excl + padoff), axis=1, keepdims=True)
    pos_ref[...] = jnp.concatenate([pos0, pos1], axis=1).astype(jnp.int32)
    gates_ref[...] = jnp.concatenate([g1, g2], axis=1)
    tend = cum8 / TILE
    tv = lax.broadcasted_iota(jnp.int32, (T_MAX, E), 0).astype(f32)
    te = jnp.sum((tv >= tend).astype(f32), axis=1, keepdims=True)
    te_ref[...] = jnp.minimum(te, E - 1).astype(jnp.int32)
    nv_ref[...] = (jnp.sum(segpad, axis=1, keepdims=True) / TILE).astype(jnp.int32)


def _routing(x, Wg):
    return pl.pallas_call(
        _route_body,
        out_shape=(
            jax.ShapeDtypeStruct((N_TOK, TOP_K), jnp.int32),
            jax.ShapeDtypeStruct((N_TOK, TOP_K), jnp.float32),
            jax.ShapeDtypeStruct((T_MAX, 1), jnp.int32),
            jax.ShapeDtypeStruct((1, 1), jnp.int32),
        ),
        scratch_shapes=[
            pltpu.VMEM((N_TOK, N_EXPERTS), jnp.float32),
            pltpu.VMEM((N_TOK, N_EXPERTS), jnp.float32),
        ],
    )(x, Wg)


_SC_MESH = plsc.VectorSubcoreMesh(core_axis_name="c", subcore_axis_name="s")


@functools.partial(
    pl.kernel,
    mesh=_SC_MESH,
    out_type=[
        jax.ShapeDtypeStruct((PAD_ROWS, D_MODEL), jnp.float32),
        jax.ShapeDtypeStruct((PAD_ROWS,), jnp.float32),
    ],
    scratch_types=[
        pltpu.VMEM((NDSUB, DSUB), jnp.int32),
        pltpu.VMEM((NDSUB, DSUB), jnp.int32),
        pltpu.VMEM((NDSUB, DSUB), jnp.float32),
        pltpu.VMEM((DSUB, D_MODEL), jnp.float32),
        pltpu.SemaphoreType.DMA,
        pltpu.SemaphoreType.DMA,
    ],
)
def _dispatch(x_hbm, pos_hbm, tok_hbm, g_hbm, xs_hbm, gs_hbm,
              pos_v, tok_v, g_v, rows_v, sem1, sem2):
    wid = lax.axis_index("s") * 2 + lax.axis_index("c")
    pltpu.sync_copy(pos_hbm.at[wid], pos_v)
    pltpu.sync_copy(tok_hbm.at[wid], tok_v)
    pltpu.sync_copy(g_hbm.at[wid], g_v)
    for j in range(NDSUB):
        pltpu.async_copy(x_hbm.at[tok_v.at[j]], rows_v, sem1).wait()
        pltpu.async_copy(rows_v, xs_hbm.at[pos_v.at[j]], sem2).wait()
        pltpu.async_copy(g_v.at[j], gs_hbm.at[pos_v.at[j]], sem2).wait()


def _ffn_pass1_body(te_ref, nv_ref, xs_ref, w1_ref, b1_ref, w2_ref, out_ref):
    t = pl.program_id(0)

    @pl.when(t < nv_ref[0])
    def _():
        h = jnp.dot(xs_ref[...], w1_ref[0], preferred_element_type=jnp.float32)
        h = jax.nn.gelu(h + b1_ref[0])
        out_ref[...] = jnp.dot(h, w2_ref[0], preferred_element_type=jnp.float32)


def _ffn_pass2_body(te_ref, nv_ref, xs_ref, w1_ref, b1_ref, w2_ref, b2_ref,
                    gs_ref, pin_ref, out_ref):
    t = pl.program_id(0)

    @pl.when(t < nv_ref[0])
    def _():
        h = jnp.dot(xs_ref[...], w1_ref[0], preferred_element_type=jnp.float32)
        h = jax.nn.gelu(h + b1_ref[0])
        o = jnp.dot(h, w2_ref[0], preferred_element_type=jnp.float32)
        o = pin_ref[...] + o + b2_ref[0]
        out_ref[...] = o * gs_ref[0, 0][:, None]


def _grouped_ffn(xs, W1, b1, W2, b2, gs, tile_expert, nvalid):
    gs3 = gs.reshape(T_MAX, 1, TILE)
    b1r = b1.reshape(N_EXPERTS, 1, D_FF)
    b2r = b2.reshape(N_EXPERTS, 1, D_MODEL)
    xs_spec = pl.BlockSpec((TILE, D_MODEL), lambda t, te, nv: (t, 0))
    out_spec = pl.BlockSpec((TILE, D_MODEL), lambda t, te, nv: (t, 0))

    def wspecs(fh):
        return [
            pl.BlockSpec((1, D_MODEL, FB), lambda t, te, nv: (te[t], 0, fh)),
            pl.BlockSpec((1, 1, FB), lambda t, te, nv: (te[t], 0, fh)),
            pl.BlockSpec((1, FB, D_MODEL), lambda t, te, nv: (te[t], fh, 0)),
        ]

    out_sds = jax.ShapeDtypeStruct((PAD_ROWS, D_MODEL), jnp.float32)
    partial1 = pl.pallas_call(
        _ffn_pass1_body,
        grid_spec=pltpu.PrefetchScalarGridSpec(
            num_scalar_prefetch=2,
            grid=(T_MAX,),
            in_specs=[xs_spec] + wspecs(0),
            out_specs=out_spec,
        ),
        out_shape=out_sds,
    )(tile_expert, nvalid, xs, W1, b1r, W2)
    return pl.pallas_call(
        _ffn_pass2_body,
        grid_spec=pltpu.PrefetchScalarGridSpec(
            num_scalar_prefetch=2,
            grid=(T_MAX,),
            in_specs=[xs_spec] + wspecs(1) + [
                pl.BlockSpec((1, 1, D_MODEL), lambda t, te, nv: (te[t], 0, 0)),
                pl.BlockSpec((1, 1, TILE), lambda t, te, nv: (t, 0, 0)),
                pl.BlockSpec((TILE, D_MODEL), lambda t, te, nv: (t, 0)),
            ],
            out_specs=out_spec,
        ),
        out_shape=out_sds,
    )(tile_expert, nvalid, xs, W1, b1r, W2, b2r, gs3, partial1)


@functools.partial(
    pl.kernel,
    mesh=_SC_MESH,
    out_type=jax.ShapeDtypeStruct((N_TOK, D_MODEL), jnp.float32),
    scratch_types=[
        pltpu.VMEM((NCSUB, CSUB), jnp.int32),
        pltpu.VMEM((NCSUB, CSUB), jnp.int32),
        pltpu.VMEM((CSUB, D_MODEL), jnp.float32),
        pltpu.VMEM((CSUB, D_MODEL), jnp.float32),
        pltpu.SemaphoreType.DMA,
        pltpu.SemaphoreType.DMA,
    ],
)
def _combine(outg_hbm, pe_hbm, po_hbm, y_hbm,
             pe_v, po_v, a_v, b_v, sem1, sem2):
    wid = lax.axis_index("s") * 2 + lax.axis_index("c")
    pltpu.sync_copy(pe_hbm.at[wid], pe_v)
    pltpu.sync_copy(po_hbm.at[wid], po_v)
    for j in range(NCSUB):
        cp1 = pltpu.async_copy(outg_hbm.at[pe_v.at[j]], a_v, sem1)
        cp2 = pltpu.async_copy(outg_hbm.at[po_v.at[j]], b_v, sem2)
        cp1.wait()
        cp2.wait()

        def col_body(c, _, r):
            base = c * 64
            for u in range(4):
                sl = pl.ds(base + u * 16, 16)
                a_v[r, sl] = a_v[r, sl] + b_v[r, sl]
            return 0

        def row_body(r, _):
            lax.fori_loop(0, D_MODEL // 64,
                          functools.partial(col_body, r=r), 0)
            return 0

        lax.fori_loop(0, CSUB, row_body, 0)
        pltpu.sync_copy(a_v, y_hbm.at[pl.ds(wid * TOKS_W + j * CSUB, CSUB)])


def kernel(x, Wg, W1, b1, W2, b2):
    pos2, gates2, te, nv = _routing(x, Wg)
    te_flat = te.reshape(T_MAX)
    nv_flat = nv.reshape(1)
    pos3 = pos2.reshape(NW, NDSUB, DSUB)
    tok3 = (jnp.arange(N_TOK * TOP_K, dtype=jnp.int32) // TOP_K).reshape(
        NW, NDSUB, DSUB)
    g3 = gates2.reshape(NW, NDSUB, DSUB)
    xs, gsort = _dispatch(x, pos3, tok3, g3)
    outg = _grouped_ffn(xs, W1, b1, W2, b2, gsort, te_flat, nv_flat)
    pe = pos2[:, 0].reshape(NW, NCSUB, CSUB)
    po = pos2[:, 1].reshape(NW, NCSUB, CSUB)
    y = _combine(outg, pe, po)
    return y


# speedup vs baseline: 1.0850x; 1.0139x over previous
import functools

import jax
import jax.numpy as jnp
from jax import lax
from jax.experimental import pallas as pl
from jax.experimental.pallas import tpu as pltpu
from jax.experimental.pallas import tpu_sc as plsc

N_TOK = 4096
D_MODEL = 1024
D_FF = 4096
N_EXPERTS = 8
TOP_K = 2
TILE = 256
T_MAX = (N_TOK * TOP_K) // TILE + N_EXPERTS
PAD_ROWS = T_MAX * TILE
FSPLIT = 2
FB = D_FF // FSPLIT

NW = 32
SLOTS_W = (N_TOK * TOP_K) // NW
DSUB = 32
NDSUB = SLOTS_W // DSUB
TOKS_W = N_TOK // NW
CSUB = 32
NCSUB = TOKS_W // CSUB


def _route_body(x_ref, wg_ref, pos_ref, gates_ref, te_ref, nv_ref,
                ohs_ref, cum_ref):
    E = N_EXPERTS
    f32 = jnp.float32
    logits = jnp.dot(x_ref[...], wg_ref[...], preferred_element_type=f32)
    m = jnp.max(logits, axis=1, keepdims=True)
    ex = jnp.exp(logits - m)
    p = ex / jnp.sum(ex, axis=1, keepdims=True)
    idx8 = lax.broadcasted_iota(jnp.int32, (N_TOK, E), 1)
    g1 = jnp.max(p, axis=1, keepdims=True)
    e1 = jnp.min(jnp.where(p >= g1, idx8, E), axis=1, keepdims=True)
    pm = jnp.where(idx8 == e1, -1.0, p)
    g2 = jnp.max(pm, axis=1, keepdims=True)
    e2 = jnp.min(jnp.where(pm >= g2, idx8, E), axis=1, keepdims=True)
    oh1 = (idx8 == e1).astype(f32)
    oh2 = (idx8 == e2).astype(f32)
    ohs_ref[...] = oh1 + oh2
    r_i = lax.broadcasted_iota(jnp.int32, (128, 128), 0)
    c_i = lax.broadcasted_iota(jnp.int32, (128, 128), 1)
    ltri = (r_i >= c_i).astype(f32)

    def body(b, carry):
        off = pl.multiple_of(b * 128, 128)
        blk = ohs_ref[pl.ds(off, 128), :]
        incl = jnp.dot(ltri, blk, preferred_element_type=f32)
        cum_ref[pl.ds(off, 128), :] = incl + carry
        return carry + jnp.sum(blk, axis=0, keepdims=True)

    lax.fori_loop(0, N_TOK // 128, body, jnp.zeros((1, E), f32))
    cum = cum_ref[...]
    ohs = ohs_ref[...]
    excl = cum - ohs
    counts = cum[N_TOK - 1:, :]
    segpad = jnp.floor((counts + (TILE - 1)) / TILE) * TILE
    tri8 = (lax.broadcasted_iota(jnp.int32, (E, E), 0)
            <= lax.broadcasted_iota(jnp.int32, (E, E), 1)).astype(f32)
    cum8 = jnp.dot(segpad, tri8, preferred_element_type=f32)
    padoff = cum8 - segpad
    pos0 = jnp.sum(oh1 * (excl + padoff), axis=1, keepdims=True)
    pos1 = jnp.sum(oh2 * (excl + padoff), axis=1, keepdims=True)
    pos_ref[...] = jnp.concatenate([pos0, pos1], axis=1).astype(jnp.int32)
    gates_ref[...] = jnp.concatenate([g1, g2], axis=1)
    tend = cum8 / TILE
    tv = lax.broadcasted_iota(jnp.int32, (T_MAX, E), 0).astype(f32)
    te = jnp.sum((tv >= tend).astype(f32), axis=1, keepdims=True)
    te_ref[...] = jnp.minimum(te, E - 1).astype(jnp.int32)
    nv_ref[...] = (jnp.sum(segpad, axis=1, keepdims=True) / TILE).astype(jnp.int32)


def _routing(x, Wg):
    return pl.pallas_call(
        _route_body,
        out_shape=(
            jax.ShapeDtypeStruct((N_TOK, TOP_K), jnp.int32),
            jax.ShapeDtypeStruct((N_TOK, TOP_K), jnp.float32),
            jax.ShapeDtypeStruct((T_MAX, 1), jnp.int32),
            jax.ShapeDtypeStruct((1, 1), jnp.int32),
        ),
        scratch_shapes=[
            pltpu.VMEM((N_TOK, N_EXPERTS), jnp.float32),
            pltpu.VMEM((N_TOK, N_EXPERTS), jnp.float32),
        ],
    )(x, Wg)


_SC_MESH = plsc.VectorSubcoreMesh(core_axis_name="c", subcore_axis_name="s")


@functools.partial(
    pl.kernel,
    mesh=_SC_MESH,
    out_type=[
        jax.ShapeDtypeStruct((PAD_ROWS, D_MODEL), jnp.float32),
        jax.ShapeDtypeStruct((PAD_ROWS,), jnp.float32),
    ],
    scratch_types=[
        pltpu.VMEM((NDSUB, DSUB), jnp.int32),
        pltpu.VMEM((NDSUB, DSUB), jnp.int32),
        pltpu.VMEM((NDSUB, DSUB), jnp.float32),
        pltpu.VMEM((DSUB, D_MODEL), jnp.float32),
        pltpu.VMEM((DSUB, D_MODEL), jnp.float32),
        pltpu.SemaphoreType.DMA,
        pltpu.SemaphoreType.DMA,
        pltpu.SemaphoreType.DMA,
    ],
)
def _dispatch(x_hbm, pos_hbm, tok_hbm, g_hbm, xs_hbm, gs_hbm,
              pos_v, tok_v, g_v, rows_a, rows_b, gsem, ssem, gatesem):
    wid = lax.axis_index("s") * 2 + lax.axis_index("c")
    pltpu.sync_copy(pos_hbm.at[wid], pos_v)
    pltpu.sync_copy(tok_hbm.at[wid], tok_v)
    pltpu.sync_copy(g_hbm.at[wid], g_v)
    bufs = (rows_a, rows_b)
    gathers, scatters, gsc = {}, {}, []
    gathers[0] = pltpu.async_copy(x_hbm.at[tok_v.at[0]], bufs[0], gsem)
    for j in range(NDSUB):
        b = bufs[j % 2]
        gathers.pop(j).wait()
        scatters[j] = pltpu.async_copy(b, xs_hbm.at[pos_v.at[j]], ssem)
        gsc.append(pltpu.async_copy(g_v.at[j], gs_hbm.at[pos_v.at[j]],
                                    gatesem))
        if j + 1 < NDSUB:
            if j >= 1:
                scatters.pop(j - 1).wait()
            gathers[j + 1] = pltpu.async_copy(x_hbm.at[tok_v.at[j + 1]],
                                              bufs[(j + 1) % 2], gsem)
    for j in sorted(scatters):
        scatters.pop(j).wait()
    for h in gsc:
        h.wait()


def _ffn_pass1_body(te_ref, nv_ref, xs_ref, w1_ref, b1_ref, w2_ref, out_ref):
    t = pl.program_id(0)

    @pl.when(t < nv_ref[0])
    def _():
        h = jnp.dot(xs_ref[...], w1_ref[0], preferred_element_type=jnp.float32)
        h = jax.nn.gelu(h + b1_ref[0])
        out_ref[...] = jnp.dot(h, w2_ref[0], preferred_element_type=jnp.float32)


def _ffn_pass2_body(te_ref, nv_ref, xs_ref, w1_ref, b1_ref, w2_ref, b2_ref,
                    gs_ref, pin_ref, out_ref):
    t = pl.program_id(0)

    @pl.when(t < nv_ref[0])
    def _():
        h = jnp.dot(xs_ref[...], w1_ref[0], preferred_element_type=jnp.float32)
        h = jax.nn.gelu(h + b1_ref[0])
        o = jnp.dot(h, w2_ref[0], preferred_element_type=jnp.float32)
        o = pin_ref[...] + o + b2_ref[0]
        out_ref[...] = o * gs_ref[0, 0][:, None]


def _grouped_ffn(xs, W1, b1, W2, b2, gs, tile_expert, nvalid):
    gs3 = gs.reshape(T_MAX, 1, TILE)
    b1r = b1.reshape(N_EXPERTS, 1, D_FF)
    b2r = b2.reshape(N_EXPERTS, 1, D_MODEL)
    xs_spec = pl.BlockSpec((TILE, D_MODEL), lambda t, te, nv: (t, 0))
    out_spec = pl.BlockSpec((TILE, D_MODEL), lambda t, te, nv: (t, 0))

    def wspecs(fh):
        return [
            pl.BlockSpec((1, D_MODEL, FB), lambda t, te, nv: (te[t], 0, fh)),
            pl.BlockSpec((1, 1, FB), lambda t, te, nv: (te[t], 0, fh)),
            pl.BlockSpec((1, FB, D_MODEL), lambda t, te, nv: (te[t], fh, 0)),
        ]

    out_sds = jax.ShapeDtypeStruct((PAD_ROWS, D_MODEL), jnp.float32)
    partial1 = pl.pallas_call(
        _ffn_pass1_body,
        grid_spec=pltpu.PrefetchScalarGridSpec(
            num_scalar_prefetch=2,
            grid=(T_MAX,),
            in_specs=[xs_spec] + wspecs(0),
            out_specs=out_spec,
        ),
        out_shape=out_sds,
    )(tile_expert, nvalid, xs, W1, b1r, W2)
    return pl.pallas_call(
        _ffn_pass2_body,
        grid_spec=pltpu.PrefetchScalarGridSpec(
            num_scalar_prefetch=2,
            grid=(T_MAX,),
            in_specs=[xs_spec] + wspecs(1) + [
                pl.BlockSpec((1, 1, D_MODEL), lambda t, te, nv: (te[t], 0, 0)),
                pl.BlockSpec((1, 1, TILE), lambda t, te, nv: (t, 0, 0)),
                pl.BlockSpec((TILE, D_MODEL), lambda t, te, nv: (t, 0)),
            ],
            out_specs=out_spec,
        ),
        out_shape=out_sds,
    )(tile_expert, nvalid, xs, W1, b1r, W2, b2r, gs3, partial1)


@functools.partial(
    pl.kernel,
    mesh=_SC_MESH,
    out_type=jax.ShapeDtypeStruct((N_TOK, D_MODEL), jnp.float32),
    scratch_types=[
        pltpu.VMEM((NCSUB, CSUB), jnp.int32),
        pltpu.VMEM((NCSUB, CSUB), jnp.int32),
        pltpu.VMEM((CSUB, D_MODEL), jnp.float32),
        pltpu.VMEM((CSUB, D_MODEL), jnp.float32),
        pltpu.SemaphoreType.DMA,
        pltpu.SemaphoreType.DMA,
    ],
)
def _combine(outg_hbm, pe_hbm, po_hbm, y_hbm,
             pe_v, po_v, a_v, b_v, sem1, sem2):
    wid = lax.axis_index("s") * 2 + lax.axis_index("c")
    pltpu.sync_copy(pe_hbm.at[wid], pe_v)
    pltpu.sync_copy(po_hbm.at[wid], po_v)
    for j in range(NCSUB):
        cp1 = pltpu.async_copy(outg_hbm.at[pe_v.at[j]], a_v, sem1)
        cp2 = pltpu.async_copy(outg_hbm.at[po_v.at[j]], b_v, sem2)
        cp1.wait()
        cp2.wait()

        def col_body(c, _, r):
            base = c * 64
            for u in range(4):
                sl = pl.ds(base + u * 16, 16)
                a_v[r, sl] = a_v[r, sl] + b_v[r, sl]
            return 0

        def row_body(r, _):
            lax.fori_loop(0, D_MODEL // 64,
                          functools.partial(col_body, r=r), 0)
            return 0

        lax.fori_loop(0, CSUB, row_body, 0)
        pltpu.sync_copy(a_v, y_hbm.at[pl.ds(wid * TOKS_W + j * CSUB, CSUB)])


def kernel(x, Wg, W1, b1, W2, b2):
    pos2, gates2, te, nv = _routing(x, Wg)
    te_flat = te.reshape(T_MAX)
    nv_flat = nv.reshape(1)
    pos3 = pos2.reshape(NW, NDSUB, DSUB)
    tok3 = (jnp.arange(N_TOK * TOP_K, dtype=jnp.int32) // TOP_K).reshape(
        NW, NDSUB, DSUB)
    g3 = gates2.reshape(NW, NDSUB, DSUB)
    xs, gsort = _dispatch(x, pos3, tok3, g3)
    outg = _grouped_ffn(xs, W1, b1, W2, b2, gsort, te_flat, nv_flat)
    pe = pos2[:, 0].reshape(NW, NCSUB, CSUB)
    po = pos2[:, 1].reshape(NW, NCSUB, CSUB)
    y = _combine(outg, pe, po)
    return y


# speedup vs baseline: 1.1334x; 1.0446x over previous
import functools

import jax
import jax.numpy as jnp
from jax import lax
from jax.experimental import pallas as pl
from jax.experimental.pallas import tpu as pltpu
from jax.experimental.pallas import tpu_sc as plsc

N_TOK = 4096
D_MODEL = 1024
D_FF = 4096
N_EXPERTS = 8
TOP_K = 2
TILE = 512
T_MAX = (N_TOK * TOP_K) // TILE + N_EXPERTS
PAD_ROWS = T_MAX * TILE
FSPLIT = 2
FB = D_FF // FSPLIT

NW = 32
SLOTS_W = (N_TOK * TOP_K) // NW
DSUB = 32
NDSUB = SLOTS_W // DSUB
TOKS_W = N_TOK // NW
CSUB = 32
NCSUB = TOKS_W // CSUB


def _route_body(x_ref, wg_ref, pos_ref, gates_ref, te_ref, nv_ref,
                ohs_ref, cum_ref):
    E = N_EXPERTS
    f32 = jnp.float32
    logits = jnp.dot(x_ref[...], wg_ref[...], preferred_element_type=f32)
    m = jnp.max(logits, axis=1, keepdims=True)
    ex = jnp.exp(logits - m)
    p = ex / jnp.sum(ex, axis=1, keepdims=True)
    idx8 = lax.broadcasted_iota(jnp.int32, (N_TOK, E), 1)
    g1 = jnp.max(p, axis=1, keepdims=True)
    e1 = jnp.min(jnp.where(p >= g1, idx8, E), axis=1, keepdims=True)
    pm = jnp.where(idx8 == e1, -1.0, p)
    g2 = jnp.max(pm, axis=1, keepdims=True)
    e2 = jnp.min(jnp.where(pm >= g2, idx8, E), axis=1, keepdims=True)
    oh1 = (idx8 == e1).astype(f32)
    oh2 = (idx8 == e2).astype(f32)
    ohs_ref[...] = oh1 + oh2
    r_i = lax.broadcasted_iota(jnp.int32, (128, 128), 0)
    c_i = lax.broadcasted_iota(jnp.int32, (128, 128), 1)
    ltri = (r_i >= c_i).astype(f32)

    def body(b, carry):
        off = pl.multiple_of(b * 128, 128)
        blk = ohs_ref[pl.ds(off, 128), :]
        incl = jnp.dot(ltri, blk, preferred_element_type=f32)
        cum_ref[pl.ds(off, 128), :] = incl + carry
        return carry + jnp.sum(blk, axis=0, keepdims=True)

    lax.fori_loop(0, N_TOK // 128, body, jnp.zeros((1, E), f32))
    cum = cum_ref[...]
    ohs = ohs_ref[...]
    excl = cum - ohs
    counts = cum[N_TOK - 1:, :]
    segpad = jnp.floor((counts + (TILE - 1)) / TILE) * TILE
    tri8 = (lax.broadcasted_iota(jnp.int32, (E, E), 0)
            <= lax.broadcasted_iota(jnp.int32, (E, E), 1)).astype(f32)
    cum8 = jnp.dot(segpad, tri8, preferred_element_type=f32)
    padoff = cum8 - segpad
    pos0 = jnp.sum(oh1 * (excl + padoff), axis=1, keepdims=True)
    pos1 = jnp.sum(oh2 * (excl + padoff), axis=1, keepdims=True)
    pos_ref[...] = jnp.concatenate([pos0, pos1], axis=1).astype(jnp.int32)
    gates_ref[...] = jnp.concatenate([g1, g2], axis=1)
    tend = cum8 / TILE
    tv = lax.broadcasted_iota(jnp.int32, (T_MAX, E), 0).astype(f32)
    te = jnp.sum((tv >= tend).astype(f32), axis=1, keepdims=True)
    te_ref[...] = jnp.minimum(te, E - 1).astype(jnp.int32)
    nv_ref[...] = (jnp.sum(segpad, axis=1, keepdims=True) / TILE).astype(jnp.int32)


def _routing(x, Wg):
    return pl.pallas_call(
        _route_body,
        out_shape=(
            jax.ShapeDtypeStruct((N_TOK, TOP_K), jnp.int32),
            jax.ShapeDtypeStruct((N_TOK, TOP_K), jnp.float32),
            jax.ShapeDtypeStruct((T_MAX, 1), jnp.int32),
            jax.ShapeDtypeStruct((1, 1), jnp.int32),
        ),
        scratch_shapes=[
            pltpu.VMEM((N_TOK, N_EXPERTS), jnp.float32),
            pltpu.VMEM((N_TOK, N_EXPERTS), jnp.float32),
        ],
    )(x, Wg)


_SC_MESH = plsc.VectorSubcoreMesh(core_axis_name="c", subcore_axis_name="s")


@functools.partial(
    pl.kernel,
    mesh=_SC_MESH,
    out_type=[
        jax.ShapeDtypeStruct((PAD_ROWS, D_MODEL), jnp.float32),
        jax.ShapeDtypeStruct((PAD_ROWS,), jnp.float32),
    ],
    scratch_types=[
        pltpu.VMEM((NDSUB, DSUB), jnp.int32),
        pltpu.VMEM((NDSUB, DSUB), jnp.int32),
        pltpu.VMEM((NDSUB, DSUB), jnp.float32),
        pltpu.VMEM((DSUB, D_MODEL), jnp.float32),
        pltpu.VMEM((DSUB, D_MODEL), jnp.float32),
        pltpu.SemaphoreType.DMA,
        pltpu.SemaphoreType.DMA,
        pltpu.SemaphoreType.DMA,
    ],
)
def _dispatch(x_hbm, pos_hbm, tok_hbm, g_hbm, xs_hbm, gs_hbm,
              pos_v, tok_v, g_v, rows_a, rows_b, gsem, ssem, gatesem):
    wid = lax.axis_index("s") * 2 + lax.axis_index("c")
    pltpu.sync_copy(pos_hbm.at[wid], pos_v)
    pltpu.sync_copy(tok_hbm.at[wid], tok_v)
    pltpu.sync_copy(g_hbm.at[wid], g_v)
    bufs = (rows_a, rows_b)
    gathers, scatters, gsc = {}, {}, []
    gathers[0] = pltpu.async_copy(x_hbm.at[tok_v.at[0]], bufs[0], gsem)
    for j in range(NDSUB):
        b = bufs[j % 2]
        gathers.pop(j).wait()
        scatters[j] = pltpu.async_copy(b, xs_hbm.at[pos_v.at[j]], ssem)
        gsc.append(pltpu.async_copy(g_v.at[j], gs_hbm.at[pos_v.at[j]],
                                    gatesem))
        if j + 1 < NDSUB:
            if j >= 1:
                scatters.pop(j - 1).wait()
            gathers[j + 1] = pltpu.async_copy(x_hbm.at[tok_v.at[j + 1]],
                                              bufs[(j + 1) % 2], gsem)
    for j in sorted(scatters):
        scatters.pop(j).wait()
    for h in gsc:
        h.wait()


def _ffn_pass1_body(te_ref, nv_ref, xs_ref, w1_ref, b1_ref, w2_ref, out_ref):
    t = pl.program_id(0)

    @pl.when(t < nv_ref[0])
    def _():
        h = jnp.dot(xs_ref[...], w1_ref[0], preferred_element_type=jnp.float32)
        h = jax.nn.gelu(h + b1_ref[0])
        out_ref[...] = jnp.dot(h, w2_ref[0], preferred_element_type=jnp.float32)


def _ffn_pass2_body(te_ref, nv_ref, xs_ref, w1_ref, b1_ref, w2_ref, b2_ref,
                    gs_ref, pin_ref, out_ref):
    t = pl.program_id(0)

    @pl.when(t < nv_ref[0])
    def _():
        h = jnp.dot(xs_ref[...], w1_ref[0], preferred_element_type=jnp.float32)
        h = jax.nn.gelu(h + b1_ref[0])
        o = jnp.dot(h, w2_ref[0], preferred_element_type=jnp.float32)
        o = pin_ref[...] + o + b2_ref[0]
        out_ref[...] = o * gs_ref[0, 0][:, None]


def _grouped_ffn(xs, W1, b1, W2, b2, gs, tile_expert, nvalid):
    gs3 = gs.reshape(T_MAX, 1, TILE)
    b1r = b1.reshape(N_EXPERTS, 1, D_FF)
    b2r = b2.reshape(N_EXPERTS, 1, D_MODEL)
    xs_spec = pl.BlockSpec((TILE, D_MODEL), lambda t, te, nv: (t, 0))
    out_spec = pl.BlockSpec((TILE, D_MODEL), lambda t, te, nv: (t, 0))

    def wspecs(fh):
        return [
            pl.BlockSpec((1, D_MODEL, FB), lambda t, te, nv: (te[t], 0, fh)),
            pl.BlockSpec((1, 1, FB), lambda t, te, nv: (te[t], 0, fh)),
            pl.BlockSpec((1, FB, D_MODEL), lambda t, te, nv: (te[t], fh, 0)),
        ]

    out_sds = jax.ShapeDtypeStruct((PAD_ROWS, D_MODEL), jnp.float32)
    partial1 = pl.pallas_call(
        _ffn_pass1_body,
        grid_spec=pltpu.PrefetchScalarGridSpec(
            num_scalar_prefetch=2,
            grid=(T_MAX,),
            in_specs=[xs_spec] + wspecs(0),
            out_specs=out_spec,
        ),
        out_shape=out_sds,
    )(tile_expert, nvalid, xs, W1, b1r, W2)
    return pl.pallas_call(
        _ffn_pass2_body,
        grid_spec=pltpu.PrefetchScalarGridSpec(
            num_scalar_prefetch=2,
            grid=(T_MAX,),
            in_specs=[xs_spec] + wspecs(1) + [
                pl.BlockSpec((1, 1, D_MODEL), lambda t, te, nv: (te[t], 0, 0)),
                pl.BlockSpec((1, 1, TILE), lambda t, te, nv: (t, 0, 0)),
                pl.BlockSpec((TILE, D_MODEL), lambda t, te, nv: (t, 0)),
            ],
            out_specs=out_spec,
        ),
        out_shape=out_sds,
    )(tile_expert, nvalid, xs, W1, b1r, W2, b2r, gs3, partial1)


@functools.partial(
    pl.kernel,
    mesh=_SC_MESH,
    out_type=jax.ShapeDtypeStruct((N_TOK, D_MODEL), jnp.float32),
    scratch_types=[
        pltpu.VMEM((NCSUB, CSUB), jnp.int32),
        pltpu.VMEM((NCSUB, CSUB), jnp.int32),
        pltpu.VMEM((CSUB, D_MODEL), jnp.float32),
        pltpu.VMEM((CSUB, D_MODEL), jnp.float32),
        pltpu.SemaphoreType.DMA,
        pltpu.SemaphoreType.DMA,
    ],
)
def _combine(outg_hbm, pe_hbm, po_hbm, y_hbm,
             pe_v, po_v, a_v, b_v, sem1, sem2):
    wid = lax.axis_index("s") * 2 + lax.axis_index("c")
    pltpu.sync_copy(pe_hbm.at[wid], pe_v)
    pltpu.sync_copy(po_hbm.at[wid], po_v)
    for j in range(NCSUB):
        cp1 = pltpu.async_copy(outg_hbm.at[pe_v.at[j]], a_v, sem1)
        cp2 = pltpu.async_copy(outg_hbm.at[po_v.at[j]], b_v, sem2)
        cp1.wait()
        cp2.wait()

        def col_body(c, _, r):
            base = c * 64
            for u in range(4):
                sl = pl.ds(base + u * 16, 16)
                a_v[r, sl] = a_v[r, sl] + b_v[r, sl]
            return 0

        def row_body(r, _):
            lax.fori_loop(0, D_MODEL // 64,
                          functools.partial(col_body, r=r), 0)
            return 0

        lax.fori_loop(0, CSUB, row_body, 0)
        pltpu.sync_copy(a_v, y_hbm.at[pl.ds(wid * TOKS_W + j * CSUB, CSUB)])


def kernel(x, Wg, W1, b1, W2, b2):
    pos2, gates2, te, nv = _routing(x, Wg)
    te_flat = te.reshape(T_MAX)
    nv_flat = nv.reshape(1)
    pos3 = pos2.reshape(NW, NDSUB, DSUB)
    tok3 = (jnp.arange(N_TOK * TOP_K, dtype=jnp.int32) // TOP_K).reshape(
        NW, NDSUB, DSUB)
    g3 = gates2.reshape(NW, NDSUB, DSUB)
    xs, gsort = _dispatch(x, pos3, tok3, g3)
    outg = _grouped_ffn(xs, W1, b1, W2, b2, gsort, te_flat, nv_flat)
    pe = pos2[:, 0].reshape(NW, NCSUB, CSUB)
    po = pos2[:, 1].reshape(NW, NCSUB, CSUB)
    y = _combine(outg, pe, po)
    return y
